# baseline (speedup 1.0000x reference)
import functools

import jax
import jax.numpy as jnp
from jax import lax
from jax.experimental import pallas as pl
from jax.experimental.pallas import tpu as pltpu
from jax.experimental.pallas import tpu_sc as plsc

_N = 16384
_C = 64
_CP = 128
_CAT_ROWS = 101
_TXT = 1536
_NTOK = 18


def _sc_cat_gather(tables_pad, cat_idx_flat):
    info = plsc.get_sparse_core_info()
    nw = info.num_cores * info.num_subcores
    chunk = _N // nw
    sub = chunk // 2

    mesh = plsc.VectorSubcoreMesh(core_axis_name="c", subcore_axis_name="s")

    @functools.partial(
        pl.kernel,
        mesh=mesh,
        out_type=jax.ShapeDtypeStruct((8, _N, _CP), jnp.float32),
        scratch_types=[
            pltpu.VMEM((sub,), jnp.int32),
            pltpu.VMEM((sub,), jnp.int32),
            pltpu.VMEM((2, sub, _CP), jnp.float32),
            pltpu.SemaphoreType.DMA,
            pltpu.SemaphoreType.DMA,
        ],
    )
    def body(tab_hbm, idx_hbm, out_hbm, idx0_v, idx1_v, rows_v, sem0, sem1):
        sems = (sem0, sem1)
        idxs = (idx0_v, idx1_v)
        wid = lax.axis_index("s") * info.num_cores + lax.axis_index("c")
        base = wid * chunk

        def src_off(s):
            return (s // 2) * _N + base + (s % 2) * sub

        def load_idx(s, b):
            iv = idxs[b]
            pltpu.sync_copy(idx_hbm.at[pl.ds(src_off(s), sub)], iv)
            off = jnp.int32((s // 2) * _CAT_ROWS)

            def add_off(j, _):
                sl = pl.ds(j * 16, 16)
                iv[sl] = iv[sl] + off
                return 0

            lax.fori_loop(0, sub // 16, add_off, 0)

        def start_gather(b):
            return pltpu.async_copy(tab_hbm.at[idxs[b]], rows_v.at[b],
                                    sems[b])

        def store(s, b):
            pltpu.sync_copy(
                rows_v.at[b],
                out_hbm.at[s // 2, pl.ds(base + (s % 2) * sub, sub)])

        load_idx(0, 0)
        h = start_gather(0)
        for s in range(1, 16):
            b = s % 2
            load_idx(s, b)
            h2 = start_gather(b)
            h.wait()
            store(s - 1, 1 - b)
            h = h2
        h.wait()
        store(15, 1)

    return body(tables_pad, cat_idx_flat)


def _tc_dense(un, inum, ut3, it3, num_w, num_bin, num_bout,
              u_w, i_w, txt_b):
    bn = 512
    nchunks = _N // bn

    def body(ut_hbm, it_hbm, un_ref, in_ref, w_ref, bin_ref,
             bout_ref, uw_ref, iw_ref, tb_ref, out_ref,
             ut_buf, it_buf, pk, us0, us1, is0, is1):
        i = pl.program_id(0)
        usems = (us0, us1)
        isems = (is0, is1)

        def tcopy(hbm_ref, buf, sem, blk):
            return pltpu.make_async_copy(
                hbm_ref.at[pl.ds(blk * bn, bn), 0, :], buf, sem)

        @pl.when(i == 0)
        def _():
            tcopy(ut_hbm, ut_buf.at[0], usems[0], 0).start()
            tcopy(it_hbm, it_buf.at[0], isems[0], 0).start()

        nxt = i + 1
        for b in (0, 1):
            @pl.when((nxt < nchunks) & (nxt % 2 == b))
            def _(b=b):
                tcopy(ut_hbm, ut_buf.at[b], usems[b], nxt).start()
                tcopy(it_hbm, it_buf.at[b], isems[b], nxt).start()

        def num_token(src_ref, col, row, par):
            x = src_ref[:, 4 * par + col:4 * par + col + 1]
            h = x * w_ref[row:row + 1, :] + bin_ref[row:row + 1, :]
            return h * jax.nn.sigmoid(h) + bout_ref[row:row + 1, :]

        for k in range(4):
            out_ref[k] = jnp.concatenate(
                [num_token(un_ref, k, k, 0), num_token(un_ref, k, k, 1)],
                axis=1)
            out_ref[9 + k] = jnp.concatenate(
                [num_token(in_ref, k, 4 + k, 0),
                 num_token(in_ref, k, 4 + k, 1)], axis=1)

        def txt_token(buf, b, wref, bias):
            pk[:, 0:_C] = jnp.dot(buf[b], wref[...],
                                  preferred_element_type=jnp.float32) + bias
            even = pk[pl.Slice(0, bn // 2, 2), 0:_C]
            odd = pk[pl.Slice(1, bn // 2, 2), 0:_C]
            return jnp.concatenate([even, odd], axis=1)

        for b in (0, 1):
            @pl.when(i % 2 == b)
            def _(b=b):
                tcopy(ut_hbm, ut_buf.at[b], usems[b], i).wait()
                tcopy(it_hbm, it_buf.at[b], isems[b], i).wait()
                out_ref[8] = txt_token(ut_buf, b, uw_ref, tb_ref[0:1, :])
                out_ref[17] = txt_token(it_buf, b, iw_ref, tb_ref[1:2, :])

    grid = (nchunks,)
    return pl.pallas_call(
        body,
        grid=grid,
        in_specs=[
            pl.BlockSpec(memory_space=pl.ANY),
            pl.BlockSpec(memory_space=pl.ANY),
            pl.BlockSpec((bn // 2, 8), lambda i: (i, 0)),
            pl.BlockSpec((bn // 2, 8), lambda i: (i, 0)),
            pl.BlockSpec((8, _C), lambda i: (0, 0)),
            pl.BlockSpec((8, _C), lambda i: (0, 0)),
            pl.BlockSpec((8, _C), lambda i: (0, 0)),
            pl.BlockSpec((_TXT, _C), lambda i: (0, 0)),
            pl.BlockSpec((_TXT, _C), lambda i: (0, 0)),
            pl.BlockSpec((2, _C), lambda i: (0, 0)),
        ],
        out_specs=pl.BlockSpec((_NTOK, bn // 2, _CP), lambda i: (0, i, 0)),
        out_shape=jax.ShapeDtypeStruct((_NTOK, _N // 2, _CP), jnp.float32),
        scratch_shapes=[
            pltpu.VMEM((2, bn, _TXT), jnp.float32),
            pltpu.VMEM((2, bn, _TXT), jnp.float32),
            pltpu.VMEM((bn, _CP), jnp.float32),
            pltpu.SemaphoreType.DMA,
            pltpu.SemaphoreType.DMA,
            pltpu.SemaphoreType.DMA,
            pltpu.SemaphoreType.DMA,
        ],
    )(ut3, it3, un, inum, num_w, num_bin, num_bout, u_w, i_w, txt_b)


def _tc_cat_fill(x3, sc_rows):
    bf = 4096

    def cat_id(c):
        return c + 4 + 5 * (c >= 4).astype(jnp.int32)

    def body(x_ref, sc_ref, out_ref):
        del x_ref
        even = sc_ref[0, pl.Slice(0, bf // 2, 2), 0:_C]
        odd = sc_ref[0, pl.Slice(1, bf // 2, 2), 0:_C]
        out_ref[0] = jnp.concatenate([even, odd], axis=1)

    return pl.pallas_call(
        body,
        grid=(_N // bf, 8),
        in_specs=[
            pl.BlockSpec(memory_space=pl.ANY),
            pl.BlockSpec((1, bf, _CP), lambda i, c: (c, i, 0)),
        ],
        out_specs=pl.BlockSpec((1, bf // 2, _CP),
                               lambda i, c: (cat_id(c), i, 0)),
        out_shape=jax.ShapeDtypeStruct((_NTOK, _N // 2, _CP), jnp.float32),
        input_output_aliases={0: 0},
    )(x3, sc_rows)


def kernel(users_numerical, users_categorical, users_text,
           items_numerical, items_categorical, items_text, params):
    p = params
    t_emb = p["table_emb"]

    num_w, num_bin, num_bout = [], [], []
    cat_tabs = []
    txt_b = []
    for tix, t in enumerate(("users", "items")):
        for i in range(4):
            num_w.append(p[f"{t}_num{i}_W"][0])
            num_bin.append(p[f"{t}_num{i}_b"])
            num_bout.append(p[f"{t}_num{i}_col"] + t_emb[tix])
        for i in range(4):
            cat_tabs.append(p[f"{t}_cat{i}_emb"]
                            + p[f"{t}_cat{i}_col"] + t_emb[tix])
        txt_b.append(p[f"{t}_txt0_b"] + p[f"{t}_txt0_col"] + t_emb[tix])
    num_w = jnp.stack(num_w)
    num_bin = jnp.stack(num_bin)
    num_bout = jnp.stack(num_bout)
    tables = jnp.concatenate(cat_tabs, axis=0)
    tables_pad = jnp.pad(tables, ((0, 0), (0, _CP - _C)))
    txt_b = jnp.stack(txt_b)

    cat_idx_flat = jnp.concatenate(
        [users_categorical.T, items_categorical.T],
        axis=0).astype(jnp.int32).reshape(-1)

    sc_rows = _sc_cat_gather(tables_pad, cat_idx_flat)
    x3 = _tc_dense(users_numerical.reshape(_N // 2, 8),
                   items_numerical.reshape(_N // 2, 8),
                   users_text, items_text,
                   num_w, num_bin, num_bout,
                   p["users_txt0_W"], p["items_txt0_W"], txt_b)
    x3 = _tc_cat_fill(x3, sc_rows)
    x = x3.reshape(_NTOK * _N, _C)

    ar = jnp.arange(_N, dtype=jnp.int32)
    node_idxs = jnp.concatenate(
        [ar + tix * _N for tix in (0, 1) for _ in range(9)])
    col_idxs = jnp.repeat(jnp.arange(_NTOK, dtype=jnp.int32), _N)
    table_idxs = jnp.repeat(
        jnp.array([0] * 9 + [1] * 9, dtype=jnp.int32), _N)
    return (x, node_idxs, col_idxs, table_idxs)


# speedup vs baseline: 1.3031x; 1.3031x over previous
import functools

import jax
import jax.numpy as jnp
from jax import lax
from jax.experimental import pallas as pl
from jax.experimental.pallas import tpu as pltpu
from jax.experimental.pallas import tpu_sc as plsc

_N = 16384
_C = 64
_CP = 128
_CAT_ROWS = 101
_TXT = 1536
_NTOK = 18


def _sc_cat_gather(tables_pad, cat_idx_flat):
    info = plsc.get_sparse_core_info()
    nw = info.num_cores * info.num_subcores
    chunk = _N // nw
    sub = chunk // 2

    mesh = plsc.VectorSubcoreMesh(core_axis_name="c", subcore_axis_name="s")

    @functools.partial(
        pl.kernel,
        mesh=mesh,
        out_type=jax.ShapeDtypeStruct((8, _N, _CP), jnp.float32),
        scratch_types=[
            pltpu.VMEM((sub,), jnp.int32),
            pltpu.VMEM((sub,), jnp.int32),
            pltpu.VMEM((2, sub, _CP), jnp.float32),
            pltpu.SemaphoreType.DMA,
            pltpu.SemaphoreType.DMA,
        ],
    )
    def body(tab_hbm, idx_hbm, out_hbm, idx0_v, idx1_v, rows_v, sem0, sem1):
        sems = (sem0, sem1)
        idxs = (idx0_v, idx1_v)
        wid = lax.axis_index("s") * info.num_cores + lax.axis_index("c")
        base = wid * chunk

        def src_off(s):
            return (s // 2) * _N + base + (s % 2) * sub

        def load_idx(s, b):
            iv = idxs[b]
            pltpu.sync_copy(idx_hbm.at[pl.ds(src_off(s), sub)], iv)
            off = jnp.int32((s // 2) * _CAT_ROWS)

            def add_off(j, _):
                sl = pl.ds(j * 16, 16)
                iv[sl] = iv[sl] + off
                return 0

            lax.fori_loop(0, sub // 16, add_off, 0)

        def start_gather(b):
            return pltpu.async_copy(tab_hbm.at[idxs[b]], rows_v.at[b],
                                    sems[b])

        def store(s, b):
            pltpu.sync_copy(
                rows_v.at[b],
                out_hbm.at[s // 2, pl.ds(base + (s % 2) * sub, sub)])

        load_idx(0, 0)
        h = start_gather(0)
        for s in range(1, 16):
            b = s % 2
            load_idx(s, b)
            h2 = start_gather(b)
            h.wait()
            store(s - 1, 1 - b)
            h = h2
        h.wait()
        store(15, 1)

    return body(tables_pad, cat_idx_flat)


def _tc_assemble(sc_rows, un, inum, ut3, it3, num_w, num_bin, num_bout,
                 u_w, i_w, txt_b):
    bn = 512
    nchunks = _N // bn

    def body(ut_hbm, it_hbm, sc_ref, un_ref, in_ref, w_ref, bin_ref,
             bout_ref, uw_ref, iw_ref, tb_ref, out_ref,
             ut_buf, it_buf, us0, us1, is0, is1):
        i = pl.program_id(0)
        usems = (us0, us1)
        isems = (is0, is1)

        def tcopy(hbm_ref, buf, sem, blk):
            return pltpu.make_async_copy(
                hbm_ref.at[pl.ds(blk * bn, bn), 0, :], buf, sem)

        @pl.when(i == 0)
        def _():
            tcopy(ut_hbm, ut_buf.at[0], usems[0], 0).start()
            tcopy(it_hbm, it_buf.at[0], isems[0], 0).start()

        nxt = i + 1
        for b in (0, 1):
            @pl.when((nxt < nchunks) & (nxt % 2 == b))
            def _(b=b):
                tcopy(ut_hbm, ut_buf.at[b], usems[b], nxt).start()
                tcopy(it_hbm, it_buf.at[b], isems[b], nxt).start()

        def num_token(src_ref, col, row):
            x = src_ref[:, col:col + 1]
            h = x * w_ref[row:row + 1, :] + bin_ref[row:row + 1, :]
            return h * jax.nn.sigmoid(h) + bout_ref[row:row + 1, :]

        for k in range(4):
            out_ref[k] = num_token(un_ref, k, k)
            out_ref[9 + k] = num_token(in_ref, k, 4 + k)
        for c in range(4):
            out_ref[4 + c] = sc_ref[c, :, :_C]
            out_ref[13 + c] = sc_ref[4 + c, :, :_C]

        for b in (0, 1):
            @pl.when(i % 2 == b)
            def _(b=b):
                tcopy(ut_hbm, ut_buf.at[b], usems[b], i).wait()
                tcopy(it_hbm, it_buf.at[b], isems[b], i).wait()
                out_ref[8] = jnp.dot(
                    ut_buf[b], uw_ref[...],
                    preferred_element_type=jnp.float32) + tb_ref[0:1, :]
                out_ref[17] = jnp.dot(
                    it_buf[b], iw_ref[...],
                    preferred_element_type=jnp.float32) + tb_ref[1:2, :]

    grid = (nchunks,)
    return pl.pallas_call(
        body,
        grid=grid,
        in_specs=[
            pl.BlockSpec(memory_space=pl.ANY),
            pl.BlockSpec(memory_space=pl.ANY),
            pl.BlockSpec((8, bn, _CP), lambda i: (0, i, 0)),
            pl.BlockSpec((bn, 4), lambda i: (i, 0)),
            pl.BlockSpec((bn, 4), lambda i: (i, 0)),
            pl.BlockSpec((8, _C), lambda i: (0, 0)),
            pl.BlockSpec((8, _C), lambda i: (0, 0)),
            pl.BlockSpec((8, _C), lambda i: (0, 0)),
            pl.BlockSpec((_TXT, _C), lambda i: (0, 0)),
            pl.BlockSpec((_TXT, _C), lambda i: (0, 0)),
            pl.BlockSpec((2, _C), lambda i: (0, 0)),
        ],
        out_specs=pl.BlockSpec((_NTOK, bn, _C), lambda i: (0, i, 0)),
        out_shape=jax.ShapeDtypeStruct((_NTOK, _N, _C), jnp.float32),
        scratch_shapes=[
            pltpu.VMEM((2, bn, _TXT), jnp.float32),
            pltpu.VMEM((2, bn, _TXT), jnp.float32),
            pltpu.SemaphoreType.DMA,
            pltpu.SemaphoreType.DMA,
            pltpu.SemaphoreType.DMA,
            pltpu.SemaphoreType.DMA,
        ],
    )(ut3, it3, sc_rows, un, inum, num_w, num_bin, num_bout, u_w, i_w, txt_b)


def _idx_tokens():
    rows = _NTOK * _N // 128

    def body(node_ref, col_ref, table_ref):
        r = jax.lax.broadcasted_iota(jnp.int32, (rows, 128), 0)
        l = jax.lax.broadcasted_iota(jnp.int32, (rows, 128), 1)
        e = r * 128 + l
        k = e >> 14
        n = e - (k << 14)
        tix = (k >= 9).astype(jnp.int32)
        node_ref[...] = n + tix * _N
        col_ref[...] = k
        table_ref[...] = tix

    shp = jax.ShapeDtypeStruct((rows, 128), jnp.int32)
    node, col, table = pl.pallas_call(
        body,
        out_shape=(shp, shp, shp),
    )()
    flat = _NTOK * _N
    return node.reshape(flat), col.reshape(flat), table.reshape(flat)


def kernel(users_numerical, users_categorical, users_text,
           items_numerical, items_categorical, items_text, params):
    p = params
    t_emb = p["table_emb"]

    num_w, num_bin, num_bout = [], [], []
    cat_tabs = []
    txt_b = []
    for tix, t in enumerate(("users", "items")):
        for i in range(4):
            num_w.append(p[f"{t}_num{i}_W"][0])
            num_bin.append(p[f"{t}_num{i}_b"])
            num_bout.append(p[f"{t}_num{i}_col"] + t_emb[tix])
        for i in range(4):
            cat_tabs.append(p[f"{t}_cat{i}_emb"]
                            + p[f"{t}_cat{i}_col"] + t_emb[tix])
        txt_b.append(p[f"{t}_txt0_b"] + p[f"{t}_txt0_col"] + t_emb[tix])
    num_w = jnp.stack(num_w)
    num_bin = jnp.stack(num_bin)
    num_bout = jnp.stack(num_bout)
    tables = jnp.concatenate(cat_tabs, axis=0)
    tables_pad = jnp.pad(tables, ((0, 0), (0, _CP - _C)))
    txt_b = jnp.stack(txt_b)

    cat_idx_flat = jnp.concatenate(
        [users_categorical.T, items_categorical.T],
        axis=0).astype(jnp.int32).reshape(-1)

    sc_rows = _sc_cat_gather(tables_pad, cat_idx_flat)
    x3 = _tc_assemble(sc_rows, users_numerical, items_numerical,
                      users_text, items_text,
                      num_w, num_bin, num_bout,
                      p["users_txt0_W"], p["items_txt0_W"], txt_b)
    x = x3.reshape(_NTOK * _N, _C)

    node_idxs, col_idxs, table_idxs = _idx_tokens()
    return (x, node_idxs, col_idxs, table_idxs)
